# baseline (speedup 1.0000x reference)
import jax
import jax.numpy as jnp
from jax.experimental import pallas as pl
from jax.experimental.pallas import tpu as pltpu

N_BITS = 8
CLIPMIN = 1e-05
QMAX = float(2**N_BITS - 1)

BLK = 512
NBUF = 4
NC = 7


@jax.jit
def kernel(x):
    rows, cols = x.shape
    nb = rows // BLK
    nc = min(NC, max(nb - NBUF - 1, 1))
    ns = nb - nc - NBUF
    interleave = ns >= 2 * nc and nc > 0

    def target(i):
        j = i - nb
        if not interleave:
            return 2 * nb - 1 - i
        r = j - NBUF
        trip = r // 3
        rem = r % 3
        t_mid = jnp.where(rem == 2, nb - 1 - trip, ns - 1 - (2 * trip + rem))
        return jnp.where(
            j < NBUF,
            nb - nc - 1 - j,
            jnp.where(r < 3 * nc, t_mid, nb - 1 - j),
        )

    def _fetch(x_hbm, dst, sem, j):
        return pltpu.make_async_copy(x_hbm.at[pl.ds(j * BLK, BLK), :], dst, sem)

    def _body(x_hbm, o_ref, buf, cache, acc_mn, acc_mx, s_ref, sems):
        i = pl.program_id(0)

        def start_fetch(j):
            @pl.when(j >= nb - nc)
            def _():
                _fetch(x_hbm, cache.at[j - (nb - nc)], sems.at[j % NBUF], j).start()

            @pl.when(j < nb - nc)
            def _():
                _fetch(x_hbm, buf.at[j % NBUF], sems.at[j % NBUF], j).start()

        def wait_fetch(j):
            @pl.when(j >= nb - nc)
            def _():
                _fetch(x_hbm, cache.at[j - (nb - nc)], sems.at[j % NBUF], j).wait()

            @pl.when(j < nb - nc)
            def _():
                _fetch(x_hbm, buf.at[j % NBUF], sems.at[j % NBUF], j).wait()

        @pl.when(i == 0)
        def _prologue():
            for k in range(NBUF):
                start_fetch(jnp.int32(k))

        @pl.when(i < nb)
        def _reduce():
            wait_fetch(i)

            def reduce_from(src):
                b3 = src.reshape(src.shape[0] // 8, 8, src.shape[1])
                pmn = jnp.min(b3, axis=0)
                pmx = jnp.max(b3, axis=0)

                @pl.when(i == 0)
                def _init():
                    acc_mn[...] = pmn
                    acc_mx[...] = pmx

                @pl.when(i > 0)
                def _acc():
                    acc_mn[...] = jnp.minimum(acc_mn[...], pmn)
                    acc_mx[...] = jnp.maximum(acc_mx[...], pmx)

            @pl.when(i >= nb - nc)
            def _from_cache():
                reduce_from(cache[i - (nb - nc)])

            @pl.when(i < nb - nc)
            def _from_buf():
                reduce_from(buf[i % NBUF])

            @pl.when(i + NBUF < nb)
            def _next():
                start_fetch(i + NBUF)

        @pl.when(i == nb)
        def _scalars():
            xmin = jnp.minimum(jnp.min(acc_mn[...]), 0.0)
            xmax = jnp.maximum(jnp.max(acc_mx[...]), 0.0)
            scale = jnp.maximum((xmax - xmin) / QMAX, CLIPMIN)
            zp = jnp.clip(jnp.round(-xmin / scale), 0.0, QMAX)
            s_ref[0] = scale
            s_ref[1] = zp

        @pl.when(i >= nb)
        def _apply():
            t = target(i)

            @pl.when(t < nb - nc - NBUF)
            def _wait_stream():
                wait_fetch(t)

            def quant(v):
                scale = s_ref[0]
                zp = s_ref[1]
                inv = 1.0 / scale
                return jnp.clip(jnp.round(v * inv), -zp, QMAX - zp) * scale

            @pl.when(t >= nb - nc)
            def _from_cache():
                o_ref[...] = quant(cache[t - (nb - nc)])

            @pl.when(t < nb - nc)
            def _from_buf():
                o_ref[...] = quant(buf[t % NBUF])

            u = t - NBUF

            @pl.when(jnp.logical_and(u >= 0, u < nb - nc - NBUF))
            def _next():
                start_fetch(u)

    def out_map(i):
        return (jnp.where(i < nb, nb - nc - 1, target(i)), 0)

    out = pl.pallas_call(
        _body,
        grid=(2 * nb,),
        in_specs=[pl.BlockSpec(memory_space=pltpu.MemorySpace.HBM)],
        out_specs=pl.BlockSpec((BLK, cols), out_map),
        out_shape=jax.ShapeDtypeStruct((rows, cols), jnp.float32),
        scratch_shapes=[
            pltpu.VMEM((NBUF, BLK, cols), jnp.float32),
            pltpu.VMEM((nc, BLK, cols), jnp.float32),
            pltpu.VMEM((8, cols), jnp.float32),
            pltpu.VMEM((8, cols), jnp.float32),
            pltpu.SMEM((2,), jnp.float32),
            pltpu.SemaphoreType.DMA((NBUF,)),
        ],
        compiler_params=pltpu.CompilerParams(
            dimension_semantics=("arbitrary",),
        ),
    )(x)
    return out


# speedup vs baseline: 1.0017x; 1.0017x over previous
import jax
import jax.numpy as jnp
from jax.experimental import pallas as pl
from jax.experimental.pallas import tpu as pltpu

N_BITS = 8
CLIPMIN = 1e-05
QMAX = float(2**N_BITS - 1)

BLK = 512
NBUF = 3
NC = 8


@jax.jit
def kernel(x):
    rows, cols = x.shape
    nb = rows // BLK
    nc = min(NC, max(nb - NBUF - 1, 1))
    ns = nb - nc - NBUF
    interleave = ns >= 2 * nc and nc > 0

    def target(i):
        j = i - nb
        if not interleave:
            return 2 * nb - 1 - i
        r = j - NBUF
        trip = r // 3
        rem = r % 3
        t_mid = jnp.where(rem == 2, nb - 1 - trip, ns - 1 - (2 * trip + rem))
        return jnp.where(
            j < NBUF,
            nb - nc - 1 - j,
            jnp.where(r < 3 * nc, t_mid, nb - 1 - j),
        )

    def _fetch(x_hbm, dst, sem, j):
        return pltpu.make_async_copy(x_hbm.at[pl.ds(j * BLK, BLK), :], dst, sem)

    def _body(x_hbm, o_ref, buf, cache, acc_mn, acc_mx, s_ref, sems):
        i = pl.program_id(0)

        def start_fetch(j):
            @pl.when(j >= nb - nc)
            def _():
                _fetch(x_hbm, cache.at[j - (nb - nc)], sems.at[j % NBUF], j).start()

            @pl.when(j < nb - nc)
            def _():
                _fetch(x_hbm, buf.at[j % NBUF], sems.at[j % NBUF], j).start()

        def wait_fetch(j):
            @pl.when(j >= nb - nc)
            def _():
                _fetch(x_hbm, cache.at[j - (nb - nc)], sems.at[j % NBUF], j).wait()

            @pl.when(j < nb - nc)
            def _():
                _fetch(x_hbm, buf.at[j % NBUF], sems.at[j % NBUF], j).wait()

        @pl.when(i == 0)
        def _prologue():
            for k in range(NBUF):
                start_fetch(jnp.int32(k))

        @pl.when(i < nb)
        def _reduce():
            wait_fetch(i)

            def reduce_from(src):
                b3 = src.reshape(src.shape[0] // 8, 8, src.shape[1])
                pmn = jnp.min(b3, axis=0)
                pmx = jnp.max(b3, axis=0)

                @pl.when(i == 0)
                def _init():
                    acc_mn[...] = pmn
                    acc_mx[...] = pmx

                @pl.when(i > 0)
                def _acc():
                    acc_mn[...] = jnp.minimum(acc_mn[...], pmn)
                    acc_mx[...] = jnp.maximum(acc_mx[...], pmx)

            @pl.when(i >= nb - nc)
            def _from_cache():
                reduce_from(cache[i - (nb - nc)])

            @pl.when(i < nb - nc)
            def _from_buf():
                reduce_from(buf[i % NBUF])

            @pl.when(i + NBUF < nb)
            def _next():
                start_fetch(i + NBUF)

        @pl.when(i == nb)
        def _scalars():
            xmin = jnp.minimum(jnp.min(acc_mn[...]), 0.0)
            xmax = jnp.maximum(jnp.max(acc_mx[...]), 0.0)
            scale = jnp.maximum((xmax - xmin) / QMAX, CLIPMIN)
            zp = jnp.clip(jnp.round(-xmin / scale), 0.0, QMAX)
            s_ref[0] = scale
            s_ref[1] = zp

        @pl.when(i >= nb)
        def _apply():
            t = target(i)

            @pl.when(t < nb - nc - NBUF)
            def _wait_stream():
                wait_fetch(t)

            def quant(v):
                scale = s_ref[0]
                zp = s_ref[1]
                inv = 1.0 / scale
                return jnp.clip(jnp.round(v * inv), -zp, QMAX - zp) * scale

            @pl.when(t >= nb - nc)
            def _from_cache():
                o_ref[...] = quant(cache[t - (nb - nc)])

            @pl.when(t < nb - nc)
            def _from_buf():
                o_ref[...] = quant(buf[t % NBUF])

            u = t - NBUF

            @pl.when(jnp.logical_and(u >= 0, u < nb - nc - NBUF))
            def _next():
                start_fetch(u)

    def out_map(i):
        return (jnp.where(i < nb, nb - nc - 1, target(i)), 0)

    out = pl.pallas_call(
        _body,
        grid=(2 * nb,),
        in_specs=[pl.BlockSpec(memory_space=pltpu.MemorySpace.HBM)],
        out_specs=pl.BlockSpec((BLK, cols), out_map),
        out_shape=jax.ShapeDtypeStruct((rows, cols), jnp.float32),
        scratch_shapes=[
            pltpu.VMEM((NBUF, BLK, cols), jnp.float32),
            pltpu.VMEM((nc, BLK, cols), jnp.float32),
            pltpu.VMEM((8, cols), jnp.float32),
            pltpu.VMEM((8, cols), jnp.float32),
            pltpu.SMEM((2,), jnp.float32),
            pltpu.SemaphoreType.DMA((NBUF,)),
        ],
        compiler_params=pltpu.CompilerParams(
            dimension_semantics=("arbitrary",),
        ),
    )(x)
    return out
